# baseline (device time: 48984 ns/iter reference)
import functools

import jax
import jax.numpy as jnp
from jax import lax
from jax.experimental import pallas as pl
from jax.experimental.pallas import tpu as pltpu

N_DEV = 16
NP = 4
MC = 256
HF = 128
HC = 512


def kernel(A, B):
    M, _ = A.shape
    _, N = B.shape

    f32 = jnp.float32
    bf16 = jnp.bfloat16

    def body(
        a_ref, b_ref, out_ref,
        p_ref, sb_cw, sb_ccw, rb_cw, rb_ccw,
        zsa_l, zsa_r, zra_l, zra_r, zsb, zrb,
        snd_cw1, snd_cw2, snd_ccw1, snd_ccw2,
        rcv_cw1, rcv_cw2, rcv_ccw1, rcv_ccw2,
        z_snd, z_rcv,
        asnd_cw, asnd_ccw, arcv_cw, arcv_ccw,
    ):
        my = lax.axis_index("i")
        qi = lax.rem(my, NP)
        zi = lax.div(my, NP)
        b0 = lax.rem(zi, 2)
        b1 = lax.div(zi, 2)

        right = zi * NP + lax.rem(qi + 1, NP)
        left = zi * NP + lax.rem(qi + 3, NP)
        dz1 = (zi + 1 - 2 * b0) * NP + qi
        dz2 = (zi + 2 - 4 * b1) * NP + qi

        def xsig(sem, dev):
            pl.semaphore_signal(
                sem, inc=1, device_id=(dev,),
                device_id_type=pl.DeviceIdType.MESH,
            )

        barrier_sem = pltpu.get_barrier_semaphore()
        for nbr in (left, right, dz1, dz2):
            xsig(barrier_sem, nbr)
        pl.semaphore_wait(barrier_sem, 4)

        def rdma(src, dst, ssem, rsem, dev):
            return pltpu.make_async_remote_copy(
                src_ref=src, dst_ref=dst, send_sem=ssem, recv_sem=rsem,
                device_id=(dev,), device_id_type=pl.DeviceIdType.MESH,
            )

        a_bf = a_ref[...].astype(bf16)

        p_ref[:, pl.ds(0, HC)] = jnp.dot(
            a_bf, b_ref[:, pl.ds(0, HC)].astype(bf16),
            preferred_element_type=f32,
        )
        cw_s0 = lax.rem(qi, NP) * MC
        sb_cw[...] = p_ref[pl.ds(cw_s0, MC), pl.ds(0, HC)].astype(bf16)
        cw1 = rdma(sb_cw.at[pl.ds(0, HF), :], rb_cw.at[0, pl.ds(0, HF), :],
                   snd_cw1, rcv_cw1.at[0], right)
        cw2 = rdma(sb_cw.at[pl.ds(HF, HF), :], rb_cw.at[0, pl.ds(HF, HF), :],
                   snd_cw2, rcv_cw2.at[0], right)
        cw1.start()
        cw2.start()

        p_ref[:, pl.ds(HC, HC)] = jnp.dot(
            a_bf, b_ref[:, pl.ds(HC, HC)].astype(bf16),
            preferred_element_type=f32,
        )
        ccw_s0 = lax.rem(qi + 2, NP) * MC
        sb_ccw[...] = p_ref[pl.ds(ccw_s0, MC), pl.ds(HC, HC)].astype(bf16)
        ccw1 = rdma(sb_ccw.at[pl.ds(0, HF), :], rb_ccw.at[0, pl.ds(0, HF), :],
                    snd_ccw1, rcv_ccw1.at[0], left)
        ccw2 = rdma(sb_ccw.at[pl.ds(HF, HF), :], rb_ccw.at[0, pl.ds(HF, HF), :],
                    snd_ccw2, rcv_ccw2.at[0], left)
        ccw1.start()
        ccw2.start()

        base = lax.rem(qi + 1, NP) * MC

        for h in range(NP - 2):
            cw_r = lax.rem(qi - h + 3, NP) * MC
            ccw_r = lax.rem(qi + h + 3, NP) * MC

            def fwd(desc, sb, rb, rows, c0, off, ssem, rsems, dev):
                desc.wait_recv()
                desc.wait_send()
                sb[pl.ds(off, HF), :] = (
                    p_ref[pl.ds(rows + off, HF), pl.ds(c0, HC)]
                    + rb[h, pl.ds(off, HF), :].astype(f32)
                ).astype(bf16)
                nxt = rdma(sb.at[pl.ds(off, HF), :],
                           rb.at[h + 1, pl.ds(off, HF), :],
                           ssem, rsems.at[h + 1], dev)
                nxt.start()
                return nxt

            cw1 = fwd(cw1, sb_cw, rb_cw, cw_r, 0, 0, snd_cw1, rcv_cw1, right)
            ccw1 = fwd(ccw1, sb_ccw, rb_ccw, ccw_r, HC, 0, snd_ccw1,
                       rcv_ccw1, left)
            cw2 = fwd(cw2, sb_cw, rb_cw, cw_r, 0, HF, snd_cw2, rcv_cw2, right)
            ccw2 = fwd(ccw2, sb_ccw, rb_ccw, ccw_r, HC, HF, snd_ccw2,
                       rcv_ccw2, left)

        keep_a = base + 128 * b0
        send_a = base + 128 * (1 - b0)
        H = NP - 2

        cw1.wait_recv()
        cw2.wait_recv()
        zsa_l[...] = (
            p_ref[pl.ds(send_a, HF), pl.ds(0, HC)]
            + rb_cw[H, pl.ds(128 * (1 - b0), HF), :].astype(f32)
        ).astype(bf16)
        za_l = rdma(zsa_l, zra_l, z_snd.at[0], z_rcv.at[0], dz1)
        za_l.start()
        p_ref[pl.ds(keep_a, HF), pl.ds(0, HC)] += (
            rb_cw[H, pl.ds(128 * b0, HF), :].astype(f32)
        )

        ccw1.wait_recv()
        ccw2.wait_recv()
        zsa_r[...] = (
            p_ref[pl.ds(send_a, HF), pl.ds(HC, HC)]
            + rb_ccw[H, pl.ds(128 * (1 - b0), HF), :].astype(f32)
        ).astype(bf16)
        za_r = rdma(zsa_r, zra_r, z_snd.at[1], z_rcv.at[1], dz1)
        za_r.start()
        p_ref[pl.ds(keep_a, HF), pl.ds(HC, HC)] += (
            rb_ccw[H, pl.ds(128 * b0, HF), :].astype(f32)
        )

        keep_b = keep_a + 64 * b1
        send_b = keep_a + 64 * (1 - b1)
        za_l.wait_recv()
        za_r.wait_recv()
        zsb[:, pl.ds(0, HC)] = (
            p_ref[pl.ds(send_b, 64), pl.ds(0, HC)]
            + zra_l[pl.ds(64 * (1 - b1), 64), :].astype(f32)
        ).astype(bf16)
        zsb[:, pl.ds(HC, HC)] = (
            p_ref[pl.ds(send_b, 64), pl.ds(HC, HC)]
            + zra_r[pl.ds(64 * (1 - b1), 64), :].astype(f32)
        ).astype(bf16)
        zb = rdma(zsb, zrb, z_snd.at[2], z_rcv.at[2], dz2)
        zb.start()
        p_ref[pl.ds(keep_b, 64), pl.ds(0, HC)] += (
            zra_l[pl.ds(64 * b1, 64), :].astype(f32)
        )
        p_ref[pl.ds(keep_b, 64), pl.ds(HC, HC)] += (
            zra_r[pl.ds(64 * b1, 64), :].astype(f32)
        )
        zb.wait_recv()
        out_ref[pl.ds(keep_b, 64), :] = (
            p_ref[pl.ds(keep_b, 64), :] + zrb[...].astype(f32)
        ).astype(bf16)

        def ag(rows, off, nr, c0, ssem, rsem, dev):
            d = rdma(out_ref.at[pl.ds(rows + off, nr), pl.ds(c0, HC)],
                     out_ref.at[pl.ds(rows + off, nr), pl.ds(c0, HC)],
                     ssem, rsem, dev)
            d.start()
            return d

        o1 = 128 * b0 + 64 * b1
        o2 = 128 * b0 + 64 * (1 - b1)
        off2 = 128 * (1 - b0)
        PIECES = ((o1, 64), (o2, 64), (off2, HF))

        acw = [None, None, None]
        accw = [None, None, None]
        acw[0] = ag(base, o1, 64, 0, asnd_cw.at[0], arcv_cw.at[0, 0], right)
        accw[0] = ag(base, o1, 64, HC, asnd_ccw.at[0], arcv_ccw.at[0, 0], left)

        zc = rdma(out_ref.at[pl.ds(keep_b, 64), :],
                  out_ref.at[pl.ds(keep_b, 64), :],
                  z_snd.at[3], z_rcv.at[3], dz2)
        zc.start()
        zc.wait_recv()

        acw[1] = ag(base, o2, 64, 0, asnd_cw.at[1], arcv_cw.at[1, 0], right)
        accw[1] = ag(base, o2, 64, HC, asnd_ccw.at[1], arcv_ccw.at[1, 0], left)

        zd = rdma(out_ref.at[pl.ds(keep_a, HF), :],
                  out_ref.at[pl.ds(keep_a, HF), :],
                  z_snd.at[4], z_rcv.at[4], dz1)
        zd.start()
        zd.wait_recv()

        acw[2] = ag(base, off2, HF, 0, asnd_cw.at[2], arcv_cw.at[2, 0], right)
        accw[2] = ag(base, off2, HF, HC, asnd_ccw.at[2], arcv_ccw.at[2, 0],
                     left)

        for h in range(NP - 1):
            n_cw = lax.rem(qi - h + NP, NP) * MC
            n_ccw = lax.rem(qi + h + 2, NP) * MC
            for k, (off, nr) in enumerate(PIECES):
                acw[k].wait_recv()
                accw[k].wait_recv()
                if h < NP - 2:
                    acw[k].wait_send()
                    acw[k] = ag(n_cw, off, nr, 0, asnd_cw.at[k],
                                arcv_cw.at[k, h + 1], right)
                    accw[k].wait_send()
                    accw[k] = ag(n_ccw, off, nr, HC, asnd_ccw.at[k],
                                 arcv_ccw.at[k, h + 1], left)

        for d in (cw1, cw2, ccw1, ccw2, za_l, za_r, zb, zc, zd,
                  *acw, *accw):
            d.wait_send()

        @functools.partial(pl.run_scoped, exit_sem=pltpu.SemaphoreType.REGULAR)
        def _(exit_sem):
            for nbr in (left, right, dz1, dz2):
                xsig(exit_sem, nbr)
            pl.semaphore_wait(exit_sem, 4)

    dma = pltpu.SemaphoreType.DMA
    return pl.pallas_call(
        body,
        out_shape=jax.ShapeDtypeStruct((M, N), bf16),
        in_specs=[
            pl.BlockSpec(memory_space=pltpu.VMEM),
            pl.BlockSpec(memory_space=pltpu.VMEM),
        ],
        out_specs=pl.BlockSpec(memory_space=pltpu.VMEM),
        scratch_shapes=[
            pltpu.VMEM((M, N), f32),
            pltpu.VMEM((MC, HC), bf16),
            pltpu.VMEM((MC, HC), bf16),
            pltpu.VMEM((NP - 1, MC, HC), bf16),
            pltpu.VMEM((NP - 1, MC, HC), bf16),
            pltpu.VMEM((HF, HC), bf16),
            pltpu.VMEM((HF, HC), bf16),
            pltpu.VMEM((HF, HC), bf16),
            pltpu.VMEM((HF, HC), bf16),
            pltpu.VMEM((64, N), bf16),
            pltpu.VMEM((64, N), bf16),
            dma, dma, dma, dma,
            dma((NP - 1,)), dma((NP - 1,)),
            dma((NP - 1,)), dma((NP - 1,)),
            dma((5,)), dma((5,)),
            dma((3,)), dma((3,)),
            dma((3, NP - 1)), dma((3, NP - 1)),
        ],
        compiler_params=pltpu.CompilerParams(collective_id=0),
    )(A, B)


# device time: 48306 ns/iter; 1.0140x vs baseline; 1.0140x over previous
import functools

import jax
import jax.numpy as jnp
from jax import lax
from jax.experimental import pallas as pl
from jax.experimental.pallas import tpu as pltpu

N_DEV = 16
NP = 4
MC = 256
HF = 128
HC = 512


def kernel(A, B):
    M, _ = A.shape
    _, N = B.shape

    f32 = jnp.float32
    bf16 = jnp.bfloat16

    def body(
        a_ref, b_ref, out_ref,
        p_ref, bb, sb_cw, sb_ccw, rb_cw, rb_ccw,
        zsa_l, zsa_r, zra_l, zra_r, zsb, zrb,
        snd_cw1, snd_cw2, snd_ccw1, snd_ccw2,
        rcv_cw1, rcv_cw2, rcv_ccw1, rcv_ccw2,
        z_snd, z_rcv,
        asnd_cw, asnd_ccw, arcv_cw, arcv_ccw,
    ):
        my = lax.axis_index("i")
        qi = lax.rem(my, NP)
        zi = lax.div(my, NP)
        b0 = lax.rem(zi, 2)
        b1 = lax.div(zi, 2)

        right = zi * NP + lax.rem(qi + 1, NP)
        left = zi * NP + lax.rem(qi + 3, NP)
        dz1 = (zi + 1 - 2 * b0) * NP + qi
        dz2 = (zi + 2 - 4 * b1) * NP + qi

        def xsig(sem, dev):
            pl.semaphore_signal(
                sem, inc=1, device_id=(dev,),
                device_id_type=pl.DeviceIdType.MESH,
            )

        barrier_sem = pltpu.get_barrier_semaphore()
        for nbr in (left, right, dz1, dz2):
            xsig(barrier_sem, nbr)
        pl.semaphore_wait(barrier_sem, 4)

        def rdma(src, dst, ssem, rsem, dev):
            return pltpu.make_async_remote_copy(
                src_ref=src, dst_ref=dst, send_sem=ssem, recv_sem=rsem,
                device_id=(dev,), device_id_type=pl.DeviceIdType.MESH,
            )

        def mm(ck, c0):
            rows = lax.rem(ck + NP, NP) * MC
            d = jnp.dot(
                a_ref[pl.ds(rows, MC), :].astype(bf16),
                bb[:, pl.ds(c0, HC)],
                preferred_element_type=f32,
            )
            p_ref[pl.ds(rows, MC), pl.ds(c0, HC)] = d
            return d

        bb[:, pl.ds(0, HC)] = b_ref[:, pl.ds(0, HC)].astype(bf16)
        sb_cw[...] = mm(qi, 0).astype(bf16)
        cw1 = rdma(sb_cw.at[pl.ds(0, HF), :], rb_cw.at[0, pl.ds(0, HF), :],
                   snd_cw1, rcv_cw1.at[0], right)
        cw2 = rdma(sb_cw.at[pl.ds(HF, HF), :], rb_cw.at[0, pl.ds(HF, HF), :],
                   snd_cw2, rcv_cw2.at[0], right)
        cw1.start()
        cw2.start()

        bb[:, pl.ds(HC, HC)] = b_ref[:, pl.ds(HC, HC)].astype(bf16)
        sb_ccw[...] = mm(qi + 2, HC).astype(bf16)
        ccw1 = rdma(sb_ccw.at[pl.ds(0, HF), :], rb_ccw.at[0, pl.ds(0, HF), :],
                    snd_ccw1, rcv_ccw1.at[0], left)
        ccw2 = rdma(sb_ccw.at[pl.ds(HF, HF), :], rb_ccw.at[0, pl.ds(HF, HF), :],
                    snd_ccw2, rcv_ccw2.at[0], left)
        ccw1.start()
        ccw2.start()

        mm(qi - 1, 0)
        mm(qi + 3, HC)
        mm(qi - 2, 0)
        mm(qi, HC)
        mm(qi + 1, 0)
        mm(qi + 1, HC)

        base = lax.rem(qi + 1, NP) * MC

        for h in range(NP - 2):
            cw_r = lax.rem(qi - h + 3, NP) * MC
            ccw_r = lax.rem(qi + h + 3, NP) * MC

            def fwd(desc, sb, rb, rows, c0, off, ssem, rsems, dev):
                desc.wait_recv()
                desc.wait_send()
                sb[pl.ds(off, HF), :] = (
                    p_ref[pl.ds(rows + off, HF), pl.ds(c0, HC)]
                    + rb[h, pl.ds(off, HF), :].astype(f32)
                ).astype(bf16)
                nxt = rdma(sb.at[pl.ds(off, HF), :],
                           rb.at[h + 1, pl.ds(off, HF), :],
                           ssem, rsems.at[h + 1], dev)
                nxt.start()
                return nxt

            cw1 = fwd(cw1, sb_cw, rb_cw, cw_r, 0, 0, snd_cw1, rcv_cw1, right)
            ccw1 = fwd(ccw1, sb_ccw, rb_ccw, ccw_r, HC, 0, snd_ccw1,
                       rcv_ccw1, left)
            cw2 = fwd(cw2, sb_cw, rb_cw, cw_r, 0, HF, snd_cw2, rcv_cw2, right)
            ccw2 = fwd(ccw2, sb_ccw, rb_ccw, ccw_r, HC, HF, snd_ccw2,
                       rcv_ccw2, left)

        keep_a = base + 128 * b0
        send_a = base + 128 * (1 - b0)
        H = NP - 2

        cw1.wait_recv()
        cw2.wait_recv()
        zsa_l[...] = (
            p_ref[pl.ds(send_a, HF), pl.ds(0, HC)]
            + rb_cw[H, pl.ds(128 * (1 - b0), HF), :].astype(f32)
        ).astype(bf16)
        za_l = rdma(zsa_l, zra_l, z_snd.at[0], z_rcv.at[0], dz1)
        za_l.start()
        p_ref[pl.ds(keep_a, HF), pl.ds(0, HC)] += (
            rb_cw[H, pl.ds(128 * b0, HF), :].astype(f32)
        )

        ccw1.wait_recv()
        ccw2.wait_recv()
        zsa_r[...] = (
            p_ref[pl.ds(send_a, HF), pl.ds(HC, HC)]
            + rb_ccw[H, pl.ds(128 * (1 - b0), HF), :].astype(f32)
        ).astype(bf16)
        za_r = rdma(zsa_r, zra_r, z_snd.at[1], z_rcv.at[1], dz1)
        za_r.start()
        p_ref[pl.ds(keep_a, HF), pl.ds(HC, HC)] += (
            rb_ccw[H, pl.ds(128 * b0, HF), :].astype(f32)
        )

        keep_b = keep_a + 64 * b1
        send_b = keep_a + 64 * (1 - b1)
        za_l.wait_recv()
        za_r.wait_recv()
        zsb[:, pl.ds(0, HC)] = (
            p_ref[pl.ds(send_b, 64), pl.ds(0, HC)]
            + zra_l[pl.ds(64 * (1 - b1), 64), :].astype(f32)
        ).astype(bf16)
        zsb[:, pl.ds(HC, HC)] = (
            p_ref[pl.ds(send_b, 64), pl.ds(HC, HC)]
            + zra_r[pl.ds(64 * (1 - b1), 64), :].astype(f32)
        ).astype(bf16)
        zb = rdma(zsb, zrb, z_snd.at[2], z_rcv.at[2], dz2)
        zb.start()
        p_ref[pl.ds(keep_b, 64), pl.ds(0, HC)] += (
            zra_l[pl.ds(64 * b1, 64), :].astype(f32)
        )
        p_ref[pl.ds(keep_b, 64), pl.ds(HC, HC)] += (
            zra_r[pl.ds(64 * b1, 64), :].astype(f32)
        )
        zb.wait_recv()
        out_ref[pl.ds(keep_b, 64), :] = (
            p_ref[pl.ds(keep_b, 64), :] + zrb[...].astype(f32)
        ).astype(bf16)

        def ag(rows, off, nr, c0, ssem, rsem, dev):
            d = rdma(out_ref.at[pl.ds(rows + off, nr), pl.ds(c0, HC)],
                     out_ref.at[pl.ds(rows + off, nr), pl.ds(c0, HC)],
                     ssem, rsem, dev)
            d.start()
            return d

        o1 = 128 * b0 + 64 * b1
        o2 = 128 * b0 + 64 * (1 - b1)
        off2 = 128 * (1 - b0)
        PIECES = ((o1, 64), (o2, 64), (off2, HF))

        acw = [None, None, None]
        accw = [None, None, None]
        acw[0] = ag(base, o1, 64, 0, asnd_cw.at[0], arcv_cw.at[0, 0], right)
        accw[0] = ag(base, o1, 64, HC, asnd_ccw.at[0], arcv_ccw.at[0, 0], left)

        zc = rdma(out_ref.at[pl.ds(keep_b, 64), :],
                  out_ref.at[pl.ds(keep_b, 64), :],
                  z_snd.at[3], z_rcv.at[3], dz2)
        zc.start()
        zc.wait_recv()

        acw[1] = ag(base, o2, 64, 0, asnd_cw.at[1], arcv_cw.at[1, 0], right)
        accw[1] = ag(base, o2, 64, HC, asnd_ccw.at[1], arcv_ccw.at[1, 0], left)

        zd = rdma(out_ref.at[pl.ds(keep_a, HF), :],
                  out_ref.at[pl.ds(keep_a, HF), :],
                  z_snd.at[4], z_rcv.at[4], dz1)
        zd.start()
        zd.wait_recv()

        acw[2] = ag(base, off2, HF, 0, asnd_cw.at[2], arcv_cw.at[2, 0], right)
        accw[2] = ag(base, off2, HF, HC, asnd_ccw.at[2], arcv_ccw.at[2, 0],
                     left)

        for h in range(NP - 1):
            n_cw = lax.rem(qi - h + NP, NP) * MC
            n_ccw = lax.rem(qi + h + 2, NP) * MC
            for k, (off, nr) in enumerate(PIECES):
                acw[k].wait_recv()
                accw[k].wait_recv()
                if h < NP - 2:
                    acw[k].wait_send()
                    acw[k] = ag(n_cw, off, nr, 0, asnd_cw.at[k],
                                arcv_cw.at[k, h + 1], right)
                    accw[k].wait_send()
                    accw[k] = ag(n_ccw, off, nr, HC, asnd_ccw.at[k],
                                 arcv_ccw.at[k, h + 1], left)

        for d in (cw1, cw2, ccw1, ccw2, za_l, za_r, zb, zc, zd,
                  *acw, *accw):
            d.wait_send()

        @functools.partial(pl.run_scoped, exit_sem=pltpu.SemaphoreType.REGULAR)
        def _(exit_sem):
            for nbr in (left, right, dz1, dz2):
                xsig(exit_sem, nbr)
            pl.semaphore_wait(exit_sem, 4)

    dma = pltpu.SemaphoreType.DMA
    return pl.pallas_call(
        body,
        out_shape=jax.ShapeDtypeStruct((M, N), bf16),
        in_specs=[
            pl.BlockSpec(memory_space=pltpu.VMEM),
            pl.BlockSpec(memory_space=pltpu.VMEM),
        ],
        out_specs=pl.BlockSpec(memory_space=pltpu.VMEM),
        scratch_shapes=[
            pltpu.VMEM((M, N), f32),
            pltpu.VMEM((A.shape[1], N), bf16),
            pltpu.VMEM((MC, HC), bf16),
            pltpu.VMEM((MC, HC), bf16),
            pltpu.VMEM((NP - 1, MC, HC), bf16),
            pltpu.VMEM((NP - 1, MC, HC), bf16),
            pltpu.VMEM((HF, HC), bf16),
            pltpu.VMEM((HF, HC), bf16),
            pltpu.VMEM((HF, HC), bf16),
            pltpu.VMEM((HF, HC), bf16),
            pltpu.VMEM((64, N), bf16),
            pltpu.VMEM((64, N), bf16),
            dma, dma, dma, dma,
            dma((NP - 1,)), dma((NP - 1,)),
            dma((NP - 1,)), dma((NP - 1,)),
            dma((5,)), dma((5,)),
            dma((3,)), dma((3,)),
            dma((3, NP - 1)), dma((3, NP - 1)),
        ],
        compiler_params=pltpu.CompilerParams(collective_id=0),
    )(A, B)


# device time: 45825 ns/iter; 1.0689x vs baseline; 1.0541x over previous
import functools

import jax
import jax.numpy as jnp
from jax import lax
from jax.experimental import pallas as pl
from jax.experimental.pallas import tpu as pltpu

N_DEV = 16
NP = 4
MC = 256
HF = 128
HC = 512


def kernel(A, B):
    M, _ = A.shape
    _, N = B.shape

    f32 = jnp.float32
    bf16 = jnp.bfloat16

    def body(
        a_ref, b_ref, out_ref,
        p_ref, bb, sb_cw, sb_ccw, rb_cw, rb_ccw,
        zsa_l, zsa_r, zra_l, zra_r, zsb, zrb,
        snd_cw1, snd_cw2, snd_ccw1, snd_ccw2,
        rcv_cw1, rcv_cw2, rcv_ccw1, rcv_ccw2,
        z_snd, z_rcv,
        asnd_cw, asnd_ccw, arcv_cw, arcv_ccw,
    ):
        my = lax.axis_index("i")
        qi = lax.rem(my, NP)
        zi = lax.div(my, NP)
        b0 = lax.rem(zi, 2)
        b1 = lax.div(zi, 2)

        right = zi * NP + lax.rem(qi + 1, NP)
        left = zi * NP + lax.rem(qi + 3, NP)
        dz1 = (zi + 1 - 2 * b0) * NP + qi
        dz2 = (zi + 2 - 4 * b1) * NP + qi

        def xsig(sem, dev):
            pl.semaphore_signal(
                sem, inc=1, device_id=(dev,),
                device_id_type=pl.DeviceIdType.MESH,
            )

        barrier_sem = pltpu.get_barrier_semaphore()
        for nbr in (left, right, dz1, dz2):
            xsig(barrier_sem, nbr)
        pl.semaphore_wait(barrier_sem, 4)

        def rdma(src, dst, ssem, rsem, dev):
            return pltpu.make_async_remote_copy(
                src_ref=src, dst_ref=dst, send_sem=ssem, recv_sem=rsem,
                device_id=(dev,), device_id_type=pl.DeviceIdType.MESH,
            )

        def mm(ck, c0):
            rows = lax.rem(ck + NP, NP) * MC
            d = jnp.dot(
                a_ref[pl.ds(rows, MC), :].astype(bf16),
                bb[:, pl.ds(c0, HC)],
                preferred_element_type=f32,
            )
            p_ref[pl.ds(rows, MC), pl.ds(c0, HC)] = d
            return d

        bb[:, pl.ds(0, HC)] = b_ref[:, pl.ds(0, HC)].astype(bf16)
        sb_cw[...] = mm(qi, 0).astype(bf16)
        cw1 = rdma(sb_cw.at[pl.ds(0, HF), :], rb_cw.at[0, pl.ds(0, HF), :],
                   snd_cw1, rcv_cw1.at[0], right)
        cw2 = rdma(sb_cw.at[pl.ds(HF, HF), :], rb_cw.at[0, pl.ds(HF, HF), :],
                   snd_cw2, rcv_cw2.at[0], right)
        cw1.start()
        cw2.start()

        bb[:, pl.ds(HC, HC)] = b_ref[:, pl.ds(HC, HC)].astype(bf16)
        sb_ccw[...] = mm(qi + 2, HC).astype(bf16)
        ccw1 = rdma(sb_ccw.at[pl.ds(0, HF), :], rb_ccw.at[0, pl.ds(0, HF), :],
                    snd_ccw1, rcv_ccw1.at[0], left)
        ccw2 = rdma(sb_ccw.at[pl.ds(HF, HF), :], rb_ccw.at[0, pl.ds(HF, HF), :],
                    snd_ccw2, rcv_ccw2.at[0], left)
        ccw1.start()
        ccw2.start()

        mm(qi - 1, 0)
        mm(qi + 3, HC)
        mm(qi - 2, 0)
        mm(qi, HC)
        mm(qi + 1, 0)
        mm(qi + 1, HC)

        base = lax.rem(qi + 1, NP) * MC

        for h in range(NP - 2):
            cw_r = lax.rem(qi - h + 3, NP) * MC
            ccw_r = lax.rem(qi + h + 3, NP) * MC

            def fwd(desc, sb, rb, rows, c0, off, ssem, rsems, dev):
                desc.wait_recv()
                desc.wait_send()
                sb[pl.ds(off, HF), :] = (
                    p_ref[pl.ds(rows + off, HF), pl.ds(c0, HC)]
                    + rb[h, pl.ds(off, HF), :].astype(f32)
                ).astype(bf16)
                nxt = rdma(sb.at[pl.ds(off, HF), :],
                           rb.at[h + 1, pl.ds(off, HF), :],
                           ssem, rsems.at[h + 1], dev)
                nxt.start()
                return nxt

            cw1 = fwd(cw1, sb_cw, rb_cw, cw_r, 0, 0, snd_cw1, rcv_cw1, right)
            ccw1 = fwd(ccw1, sb_ccw, rb_ccw, ccw_r, HC, 0, snd_ccw1,
                       rcv_ccw1, left)
            cw2 = fwd(cw2, sb_cw, rb_cw, cw_r, 0, HF, snd_cw2, rcv_cw2, right)
            ccw2 = fwd(ccw2, sb_ccw, rb_ccw, ccw_r, HC, HF, snd_ccw2,
                       rcv_ccw2, left)

        keep_a = base + 128 * b0
        send_a = base + 128 * (1 - b0)
        H = NP - 2

        cw1.wait_recv()
        cw2.wait_recv()
        zsa_l[...] = (
            p_ref[pl.ds(send_a, HF), pl.ds(0, HC)]
            + rb_cw[H, pl.ds(128 * (1 - b0), HF), :].astype(f32)
        ).astype(bf16)
        za_l = rdma(zsa_l, zra_l, z_snd.at[0], z_rcv.at[0], dz1)
        za_l.start()
        p_ref[pl.ds(keep_a, HF), pl.ds(0, HC)] += (
            rb_cw[H, pl.ds(128 * b0, HF), :].astype(f32)
        )

        ccw1.wait_recv()
        ccw2.wait_recv()
        zsa_r[...] = (
            p_ref[pl.ds(send_a, HF), pl.ds(HC, HC)]
            + rb_ccw[H, pl.ds(128 * (1 - b0), HF), :].astype(f32)
        ).astype(bf16)
        za_r = rdma(zsa_r, zra_r, z_snd.at[1], z_rcv.at[1], dz1)
        za_r.start()
        p_ref[pl.ds(keep_a, HF), pl.ds(HC, HC)] += (
            rb_ccw[H, pl.ds(128 * b0, HF), :].astype(f32)
        )

        keep_b = keep_a + 64 * b1
        send_b = keep_a + 64 * (1 - b1)
        za_l.wait_recv()
        za_r.wait_recv()
        zsb[:, pl.ds(0, HC)] = (
            p_ref[pl.ds(send_b, 64), pl.ds(0, HC)]
            + zra_l[pl.ds(64 * (1 - b1), 64), :].astype(f32)
        ).astype(bf16)
        zsb[:, pl.ds(HC, HC)] = (
            p_ref[pl.ds(send_b, 64), pl.ds(HC, HC)]
            + zra_r[pl.ds(64 * (1 - b1), 64), :].astype(f32)
        ).astype(bf16)
        zb = rdma(zsb, zrb, z_snd.at[2], z_rcv.at[2], dz2)
        zb.start()
        p_ref[pl.ds(keep_b, 64), pl.ds(0, HC)] += (
            zra_l[pl.ds(64 * b1, 64), :].astype(f32)
        )
        p_ref[pl.ds(keep_b, 64), pl.ds(HC, HC)] += (
            zra_r[pl.ds(64 * b1, 64), :].astype(f32)
        )
        zb.wait_recv()
        out_ref[pl.ds(keep_b, 64), :] = (
            p_ref[pl.ds(keep_b, 64), :] + zrb[...].astype(f32)
        ).astype(bf16)

        def ag(rows, off, c0, ssem, rsem, dev):
            d = rdma(out_ref.at[pl.ds(rows + off, 64), pl.ds(c0, HC)],
                     out_ref.at[pl.ds(rows + off, 64), pl.ds(c0, HC)],
                     ssem, rsem, dev)
            d.start()
            return d

        o1 = 128 * b0 + 64 * b1
        o2 = 128 * b0 + 64 * (1 - b1)
        o3 = 128 * (1 - b0) + 64 * b1
        o4 = 128 * (1 - b0) + 64 * (1 - b1)
        PIECES = (o1, o2, o3, o4)

        acw = [None] * 4
        accw = [None] * 4
        acw[0] = ag(base, o1, 0, asnd_cw.at[0], arcv_cw.at[0, 0], right)
        accw[0] = ag(base, o1, HC, asnd_ccw.at[0], arcv_ccw.at[0, 0], left)

        zc = rdma(out_ref.at[pl.ds(keep_b, 64), :],
                  out_ref.at[pl.ds(keep_b, 64), :],
                  z_snd.at[3], z_rcv.at[3], dz2)
        zc.start()
        zd1 = rdma(out_ref.at[pl.ds(keep_b, 64), :],
                   out_ref.at[pl.ds(keep_b, 64), :],
                   z_snd.at[4], z_rcv.at[4], dz1)
        zd1.start()

        zc.wait_recv()
        acw[1] = ag(base, o2, 0, asnd_cw.at[1], arcv_cw.at[1, 0], right)
        accw[1] = ag(base, o2, HC, asnd_ccw.at[1], arcv_ccw.at[1, 0], left)
        zd2 = rdma(out_ref.at[pl.ds(base + o2, 64), :],
                   out_ref.at[pl.ds(base + o2, 64), :],
                   z_snd.at[5], z_rcv.at[5], dz1)
        zd2.start()

        zd1.wait_recv()
        acw[2] = ag(base, o3, 0, asnd_cw.at[2], arcv_cw.at[2, 0], right)
        accw[2] = ag(base, o3, HC, asnd_ccw.at[2], arcv_ccw.at[2, 0], left)

        zd2.wait_recv()
        acw[3] = ag(base, o4, 0, asnd_cw.at[3], arcv_cw.at[3, 0], right)
        accw[3] = ag(base, o4, HC, asnd_ccw.at[3], arcv_ccw.at[3, 0], left)

        for h in range(NP - 1):
            n_cw = lax.rem(qi - h + NP, NP) * MC
            n_ccw = lax.rem(qi + h + 2, NP) * MC
            for k, off in enumerate(PIECES):
                acw[k].wait_recv()
                accw[k].wait_recv()
                if h < NP - 2:
                    acw[k].wait_send()
                    acw[k] = ag(n_cw, off, 0, asnd_cw.at[k],
                                arcv_cw.at[k, h + 1], right)
                    accw[k].wait_send()
                    accw[k] = ag(n_ccw, off, HC, asnd_ccw.at[k],
                                 arcv_ccw.at[k, h + 1], left)

        for d in (cw1, cw2, ccw1, ccw2, za_l, za_r, zb, zc, zd1, zd2,
                  *acw, *accw):
            d.wait_send()

        @functools.partial(pl.run_scoped, exit_sem=pltpu.SemaphoreType.REGULAR)
        def _(exit_sem):
            for nbr in (left, right, dz1, dz2):
                xsig(exit_sem, nbr)
            pl.semaphore_wait(exit_sem, 4)

    dma = pltpu.SemaphoreType.DMA
    return pl.pallas_call(
        body,
        out_shape=jax.ShapeDtypeStruct((M, N), bf16),
        in_specs=[
            pl.BlockSpec(memory_space=pltpu.VMEM),
            pl.BlockSpec(memory_space=pltpu.VMEM),
        ],
        out_specs=pl.BlockSpec(memory_space=pltpu.VMEM),
        scratch_shapes=[
            pltpu.VMEM((M, N), f32),
            pltpu.VMEM((A.shape[1], N), bf16),
            pltpu.VMEM((MC, HC), bf16),
            pltpu.VMEM((MC, HC), bf16),
            pltpu.VMEM((NP - 1, MC, HC), bf16),
            pltpu.VMEM((NP - 1, MC, HC), bf16),
            pltpu.VMEM((HF, HC), bf16),
            pltpu.VMEM((HF, HC), bf16),
            pltpu.VMEM((HF, HC), bf16),
            pltpu.VMEM((HF, HC), bf16),
            pltpu.VMEM((64, N), bf16),
            pltpu.VMEM((64, N), bf16),
            dma, dma, dma, dma,
            dma((NP - 1,)), dma((NP - 1,)),
            dma((NP - 1,)), dma((NP - 1,)),
            dma((6,)), dma((6,)),
            dma((4,)), dma((4,)),
            dma((4, NP - 1)), dma((4, NP - 1)),
        ],
        compiler_params=pltpu.CompilerParams(collective_id=0),
    )(A, B)


# device time: 43352 ns/iter; 1.1299x vs baseline; 1.0570x over previous
import functools

import jax
import jax.numpy as jnp
from jax import lax
from jax.experimental import pallas as pl
from jax.experimental.pallas import tpu as pltpu

N_DEV = 16
NP = 4
MC = 256
HF = 128
HC = 512


def kernel(A, B):
    M, _ = A.shape
    _, N = B.shape

    f32 = jnp.float32
    bf16 = jnp.bfloat16

    def body(
        a_ref, b_ref, out_ref,
        p_ref, bb, sb_cw, sb_ccw, rb_cw, rb_ccw,
        zsa_l, zsa_r, zra_l, zra_r, zsb_l, zsb_r, zrb_l, zrb_r,
        snd_cw1, snd_cw2, snd_ccw1, snd_ccw2,
        rcv_cw1, rcv_cw2, rcv_ccw1, rcv_ccw2,
        z_snd, z_rcv,
        asnd_cw, asnd_ccw, arcv_cw, arcv_ccw,
    ):
        my = lax.axis_index("i")
        qi = lax.rem(my, NP)
        zi = lax.div(my, NP)
        b0 = lax.rem(zi, 2)
        b1 = lax.div(zi, 2)

        right = zi * NP + lax.rem(qi + 1, NP)
        left = zi * NP + lax.rem(qi + 3, NP)
        dz1 = (zi + 1 - 2 * b0) * NP + qi
        dz2 = (zi + 2 - 4 * b1) * NP + qi

        def xsig(sem, dev):
            pl.semaphore_signal(
                sem, inc=1, device_id=(dev,),
                device_id_type=pl.DeviceIdType.MESH,
            )

        barrier_sem = pltpu.get_barrier_semaphore()
        for nbr in (left, right, dz1, dz2):
            xsig(barrier_sem, nbr)
        pl.semaphore_wait(barrier_sem, 4)

        def rdma(src, dst, ssem, rsem, dev):
            return pltpu.make_async_remote_copy(
                src_ref=src, dst_ref=dst, send_sem=ssem, recv_sem=rsem,
                device_id=(dev,), device_id_type=pl.DeviceIdType.MESH,
            )

        def mm(ck, c0):
            rows = lax.rem(ck + NP, NP) * MC
            d = jnp.dot(
                a_ref[pl.ds(rows, MC), :].astype(bf16),
                bb[:, pl.ds(c0, HC)],
                preferred_element_type=f32,
            )
            p_ref[pl.ds(rows, MC), pl.ds(c0, HC)] = d
            return d

        bb[:, pl.ds(0, HC)] = b_ref[:, pl.ds(0, HC)].astype(bf16)
        sb_cw[...] = mm(qi, 0).astype(bf16)
        cw1 = rdma(sb_cw.at[pl.ds(0, HF), :], rb_cw.at[0, pl.ds(0, HF), :],
                   snd_cw1, rcv_cw1.at[0], right)
        cw2 = rdma(sb_cw.at[pl.ds(HF, HF), :], rb_cw.at[0, pl.ds(HF, HF), :],
                   snd_cw2, rcv_cw2.at[0], right)
        cw1.start()
        cw2.start()

        bb[:, pl.ds(HC, HC)] = b_ref[:, pl.ds(HC, HC)].astype(bf16)
        sb_ccw[...] = mm(qi + 2, HC).astype(bf16)
        ccw1 = rdma(sb_ccw.at[pl.ds(0, HF), :], rb_ccw.at[0, pl.ds(0, HF), :],
                    snd_ccw1, rcv_ccw1.at[0], left)
        ccw2 = rdma(sb_ccw.at[pl.ds(HF, HF), :], rb_ccw.at[0, pl.ds(HF, HF), :],
                    snd_ccw2, rcv_ccw2.at[0], left)
        ccw1.start()
        ccw2.start()

        mm(qi - 1, 0)
        mm(qi + 3, HC)
        mm(qi - 2, 0)
        mm(qi, HC)
        mm(qi + 1, 0)
        mm(qi + 1, HC)

        base = lax.rem(qi + 1, NP) * MC

        for h in range(NP - 2):
            cw_r = lax.rem(qi - h + 3, NP) * MC
            ccw_r = lax.rem(qi + h + 3, NP) * MC

            def fwd(desc, sb, rb, rows, c0, off, ssem, rsems, dev):
                desc.wait_recv()
                desc.wait_send()
                sb[pl.ds(off, HF), :] = (
                    p_ref[pl.ds(rows + off, HF), pl.ds(c0, HC)]
                    + rb[h, pl.ds(off, HF), :].astype(f32)
                ).astype(bf16)
                nxt = rdma(sb.at[pl.ds(off, HF), :],
                           rb.at[h + 1, pl.ds(off, HF), :],
                           ssem, rsems.at[h + 1], dev)
                nxt.start()
                return nxt

            cw1 = fwd(cw1, sb_cw, rb_cw, cw_r, 0, 0, snd_cw1, rcv_cw1, right)
            ccw1 = fwd(ccw1, sb_ccw, rb_ccw, ccw_r, HC, 0, snd_ccw1,
                       rcv_ccw1, left)
            cw2 = fwd(cw2, sb_cw, rb_cw, cw_r, 0, HF, snd_cw2, rcv_cw2, right)
            ccw2 = fwd(ccw2, sb_ccw, rb_ccw, ccw_r, HC, HF, snd_ccw2,
                       rcv_ccw2, left)

        keep_a = base + 128 * b0
        send_a = base + 128 * (1 - b0)
        H = NP - 2

        cw1.wait_recv()
        cw2.wait_recv()
        zsa_l[...] = (
            p_ref[pl.ds(send_a, HF), pl.ds(0, HC)]
            + rb_cw[H, pl.ds(128 * (1 - b0), HF), :].astype(f32)
        ).astype(bf16)
        za_l = rdma(zsa_l, zra_l, z_snd.at[0], z_rcv.at[0], dz1)
        za_l.start()
        p_ref[pl.ds(keep_a, HF), pl.ds(0, HC)] += (
            rb_cw[H, pl.ds(128 * b0, HF), :].astype(f32)
        )

        ccw1.wait_recv()
        ccw2.wait_recv()
        zsa_r[...] = (
            p_ref[pl.ds(send_a, HF), pl.ds(HC, HC)]
            + rb_ccw[H, pl.ds(128 * (1 - b0), HF), :].astype(f32)
        ).astype(bf16)
        za_r = rdma(zsa_r, zra_r, z_snd.at[1], z_rcv.at[1], dz1)
        za_r.start()
        p_ref[pl.ds(keep_a, HF), pl.ds(HC, HC)] += (
            rb_ccw[H, pl.ds(128 * b0, HF), :].astype(f32)
        )

        keep_b = keep_a + 64 * b1
        send_b = keep_a + 64 * (1 - b1)
        za_l.wait_recv()
        zsb_l[...] = (
            p_ref[pl.ds(send_b, 64), pl.ds(0, HC)]
            + zra_l[pl.ds(64 * (1 - b1), 64), :].astype(f32)
        ).astype(bf16)
        zb_l = rdma(zsb_l, zrb_l, z_snd.at[2], z_rcv.at[2], dz2)
        zb_l.start()
        p_ref[pl.ds(keep_b, 64), pl.ds(0, HC)] += (
            zra_l[pl.ds(64 * b1, 64), :].astype(f32)
        )
        za_r.wait_recv()
        zsb_r[...] = (
            p_ref[pl.ds(send_b, 64), pl.ds(HC, HC)]
            + zra_r[pl.ds(64 * (1 - b1), 64), :].astype(f32)
        ).astype(bf16)
        zb_r = rdma(zsb_r, zrb_r, z_snd.at[3], z_rcv.at[3], dz2)
        zb_r.start()
        p_ref[pl.ds(keep_b, 64), pl.ds(HC, HC)] += (
            zra_r[pl.ds(64 * b1, 64), :].astype(f32)
        )

        def ag(rows, off, c0, ssem, rsem, dev):
            d = rdma(out_ref.at[pl.ds(rows + off, 64), pl.ds(c0, HC)],
                     out_ref.at[pl.ds(rows + off, 64), pl.ds(c0, HC)],
                     ssem, rsem, dev)
            d.start()
            return d

        o1 = 128 * b0 + 64 * b1
        o2 = 128 * b0 + 64 * (1 - b1)
        o3 = 128 * (1 - b0) + 64 * b1
        o4 = 128 * (1 - b0) + 64 * (1 - b1)
        PIECES = (o1, o2, o3, o4)

        def zswap(rows, c0, k, dev):
            d = rdma(out_ref.at[pl.ds(rows, 64), pl.ds(c0, HC)],
                     out_ref.at[pl.ds(rows, 64), pl.ds(c0, HC)],
                     z_snd.at[k], z_rcv.at[k], dev)
            d.start()
            return d

        acw = [None] * 4
        accw = [None] * 4

        zb_l.wait_recv()
        out_ref[pl.ds(keep_b, 64), pl.ds(0, HC)] = (
            p_ref[pl.ds(keep_b, 64), pl.ds(0, HC)] + zrb_l[...].astype(f32)
        ).astype(bf16)
        acw[0] = ag(base, o1, 0, asnd_cw.at[0], arcv_cw.at[0, 0], right)
        zc_l = zswap(keep_b, 0, 4, dz2)
        zd1_l = zswap(keep_b, 0, 6, dz1)

        zb_r.wait_recv()
        out_ref[pl.ds(keep_b, 64), pl.ds(HC, HC)] = (
            p_ref[pl.ds(keep_b, 64), pl.ds(HC, HC)] + zrb_r[...].astype(f32)
        ).astype(bf16)
        accw[0] = ag(base, o1, HC, asnd_ccw.at[0], arcv_ccw.at[0, 0], left)
        zc_r = zswap(keep_b, HC, 5, dz2)
        zd1_r = zswap(keep_b, HC, 7, dz1)

        zc_l.wait_recv()
        acw[1] = ag(base, o2, 0, asnd_cw.at[1], arcv_cw.at[1, 0], right)
        zd2_l = zswap(base + o2, 0, 8, dz1)
        zc_r.wait_recv()
        accw[1] = ag(base, o2, HC, asnd_ccw.at[1], arcv_ccw.at[1, 0], left)
        zd2_r = zswap(base + o2, HC, 9, dz1)

        zd1_l.wait_recv()
        acw[2] = ag(base, o3, 0, asnd_cw.at[2], arcv_cw.at[2, 0], right)
        zd1_r.wait_recv()
        accw[2] = ag(base, o3, HC, asnd_ccw.at[2], arcv_ccw.at[2, 0], left)

        zd2_l.wait_recv()
        acw[3] = ag(base, o4, 0, asnd_cw.at[3], arcv_cw.at[3, 0], right)
        zd2_r.wait_recv()
        accw[3] = ag(base, o4, HC, asnd_ccw.at[3], arcv_ccw.at[3, 0], left)

        for h in range(NP - 1):
            n_cw = lax.rem(qi - h + NP, NP) * MC
            n_ccw = lax.rem(qi + h + 2, NP) * MC
            for k, off in enumerate(PIECES):
                acw[k].wait_recv()
                accw[k].wait_recv()
                if h < NP - 2:
                    acw[k].wait_send()
                    acw[k] = ag(n_cw, off, 0, asnd_cw.at[k],
                                arcv_cw.at[k, h + 1], right)
                    accw[k].wait_send()
                    accw[k] = ag(n_ccw, off, HC, asnd_ccw.at[k],
                                 arcv_ccw.at[k, h + 1], left)

        for d in (cw1, cw2, ccw1, ccw2, za_l, za_r, zb_l, zb_r,
                  zc_l, zc_r, zd1_l, zd1_r, zd2_l, zd2_r, *acw, *accw):
            d.wait_send()

        @functools.partial(pl.run_scoped, exit_sem=pltpu.SemaphoreType.REGULAR)
        def _(exit_sem):
            for nbr in (left, right, dz1, dz2):
                xsig(exit_sem, nbr)
            pl.semaphore_wait(exit_sem, 4)

    dma = pltpu.SemaphoreType.DMA
    return pl.pallas_call(
        body,
        out_shape=jax.ShapeDtypeStruct((M, N), bf16),
        in_specs=[
            pl.BlockSpec(memory_space=pltpu.VMEM),
            pl.BlockSpec(memory_space=pltpu.VMEM),
        ],
        out_specs=pl.BlockSpec(memory_space=pltpu.VMEM),
        scratch_shapes=[
            pltpu.VMEM((M, N), f32),
            pltpu.VMEM((A.shape[1], N), bf16),
            pltpu.VMEM((MC, HC), bf16),
            pltpu.VMEM((MC, HC), bf16),
            pltpu.VMEM((NP - 1, MC, HC), bf16),
            pltpu.VMEM((NP - 1, MC, HC), bf16),
            pltpu.VMEM((HF, HC), bf16),
            pltpu.VMEM((HF, HC), bf16),
            pltpu.VMEM((HF, HC), bf16),
            pltpu.VMEM((HF, HC), bf16),
            pltpu.VMEM((64, HC), bf16),
            pltpu.VMEM((64, HC), bf16),
            pltpu.VMEM((64, HC), bf16),
            pltpu.VMEM((64, HC), bf16),
            dma, dma, dma, dma,
            dma((NP - 1,)), dma((NP - 1,)),
            dma((NP - 1,)), dma((NP - 1,)),
            dma((10,)), dma((10,)),
            dma((4,)), dma((4,)),
            dma((4, NP - 1)), dma((4, NP - 1)),
        ],
        compiler_params=pltpu.CompilerParams(collective_id=0),
    )(A, B)


# device time: 41138 ns/iter; 1.1907x vs baseline; 1.0538x over previous
import functools

import jax
import jax.numpy as jnp
from jax import lax
from jax.experimental import pallas as pl
from jax.experimental.pallas import tpu as pltpu

N_DEV = 16
NP = 4
MC = 256
HF = 128
HC = 512


def kernel(A, B):
    M, _ = A.shape
    _, N = B.shape

    f32 = jnp.float32
    bf16 = jnp.bfloat16

    def body(
        a_ref, b_ref, out_ref,
        p_ref, bb, sb_cw, sb_ccw, rb_cw, rb_ccw,
        zsa_l, zsa_r, zra_l, zra_r, zsb_l, zsb_r, zrb_l, zrb_r,
        snd_cw, snd_ccw, rcv_cw, rcv_ccw,
        z_snd, z_rcv,
        asnd_cw, asnd_ccw, arcv_cw, arcv_ccw,
    ):
        my = lax.axis_index("i")
        qi = lax.rem(my, NP)
        zi = lax.div(my, NP)
        b0 = lax.rem(zi, 2)
        b1 = lax.div(zi, 2)

        right = zi * NP + lax.rem(qi + 1, NP)
        left = zi * NP + lax.rem(qi + 3, NP)
        dz1 = (zi + 1 - 2 * b0) * NP + qi
        dz2 = (zi + 2 - 4 * b1) * NP + qi

        def xsig(sem, dev):
            pl.semaphore_signal(
                sem, inc=1, device_id=(dev,),
                device_id_type=pl.DeviceIdType.MESH,
            )

        barrier_sem = pltpu.get_barrier_semaphore()
        for nbr in (left, right, dz1, dz2):
            xsig(barrier_sem, nbr)
        pl.semaphore_wait(barrier_sem, 4)

        def rdma(src, dst, ssem, rsem, dev):
            return pltpu.make_async_remote_copy(
                src_ref=src, dst_ref=dst, send_sem=ssem, recv_sem=rsem,
                device_id=(dev,), device_id_type=pl.DeviceIdType.MESH,
            )

        def mm(ck, c0):
            rows = lax.rem(ck + NP, NP) * MC
            d = jnp.dot(
                a_ref[pl.ds(rows, MC), :].astype(bf16),
                bb[:, pl.ds(c0, HC)],
                preferred_element_type=f32,
            )
            p_ref[pl.ds(rows, MC), pl.ds(c0, HC)] = d
            return d

        POFF = (
            128 * (1 - b0),
            128 * (1 - b0) + 64,
            128 * b0,
            128 * b0 + 64,
        )

        def rs_start(sb, rb, rsems, dev):
            ds = []
            for k, off in enumerate(POFF):
                d = rdma(sb.at[pl.ds(off, 64), :], rb.at[0, pl.ds(off, 64), :],
                         (snd_cw if dev is right else snd_ccw).at[k],
                         rsems.at[k, 0], dev)
                d.start()
                ds.append(d)
            return ds

        bb[:, pl.ds(0, HC)] = b_ref[:, pl.ds(0, HC)].astype(bf16)
        sb_cw[...] = mm(qi, 0).astype(bf16)
        cwd = rs_start(sb_cw, rb_cw, rcv_cw, right)

        bb[:, pl.ds(HC, HC)] = b_ref[:, pl.ds(HC, HC)].astype(bf16)
        sb_ccw[...] = mm(qi + 2, HC).astype(bf16)
        ccwd = rs_start(sb_ccw, rb_ccw, rcv_ccw, left)

        mm(qi - 1, 0)
        mm(qi + 3, HC)
        mm(qi - 2, 0)
        mm(qi, HC)
        mm(qi + 1, 0)
        mm(qi + 1, HC)

        base = lax.rem(qi + 1, NP) * MC

        for h in range(NP - 2):
            cw_r = lax.rem(qi - h + 3, NP) * MC
            ccw_r = lax.rem(qi + h + 3, NP) * MC

            def fwd(desc, sb, rb, rows, c0, k, ssems, rsems, dev):
                off = POFF[k]
                desc.wait_recv()
                desc.wait_send()
                sb[pl.ds(off, 64), :] = (
                    p_ref[pl.ds(rows + off, 64), pl.ds(c0, HC)]
                    + rb[h, pl.ds(off, 64), :].astype(f32)
                ).astype(bf16)
                nxt = rdma(sb.at[pl.ds(off, 64), :],
                           rb.at[h + 1, pl.ds(off, 64), :],
                           ssems.at[k], rsems.at[k, h + 1], dev)
                nxt.start()
                return nxt

            for k in range(4):
                cwd[k] = fwd(cwd[k], sb_cw, rb_cw, cw_r, 0, k,
                             snd_cw, rcv_cw, right)
                ccwd[k] = fwd(ccwd[k], sb_ccw, rb_ccw, ccw_r, HC, k,
                              snd_ccw, rcv_ccw, left)

        keep_a = base + 128 * b0
        send_a = base + 128 * (1 - b0)
        H = NP - 2

        cwd[0].wait_recv()
        cwd[1].wait_recv()
        zsa_l[...] = (
            p_ref[pl.ds(send_a, HF), pl.ds(0, HC)]
            + rb_cw[H, pl.ds(128 * (1 - b0), HF), :].astype(f32)
        ).astype(bf16)
        za_l = rdma(zsa_l, zra_l, z_snd.at[0], z_rcv.at[0], dz1)
        za_l.start()

        ccwd[0].wait_recv()
        ccwd[1].wait_recv()
        zsa_r[...] = (
            p_ref[pl.ds(send_a, HF), pl.ds(HC, HC)]
            + rb_ccw[H, pl.ds(128 * (1 - b0), HF), :].astype(f32)
        ).astype(bf16)
        za_r = rdma(zsa_r, zra_r, z_snd.at[1], z_rcv.at[1], dz1)
        za_r.start()

        cwd[2].wait_recv()
        cwd[3].wait_recv()
        p_ref[pl.ds(keep_a, HF), pl.ds(0, HC)] += (
            rb_cw[H, pl.ds(128 * b0, HF), :].astype(f32)
        )
        ccwd[2].wait_recv()
        ccwd[3].wait_recv()
        p_ref[pl.ds(keep_a, HF), pl.ds(HC, HC)] += (
            rb_ccw[H, pl.ds(128 * b0, HF), :].astype(f32)
        )

        keep_b = keep_a + 64 * b1
        send_b = keep_a + 64 * (1 - b1)
        za_l.wait_recv()
        zsb_l[...] = (
            p_ref[pl.ds(send_b, 64), pl.ds(0, HC)]
            + zra_l[pl.ds(64 * (1 - b1), 64), :].astype(f32)
        ).astype(bf16)
        zb_l = rdma(zsb_l, zrb_l, z_snd.at[2], z_rcv.at[2], dz2)
        zb_l.start()
        p_ref[pl.ds(keep_b, 64), pl.ds(0, HC)] += (
            zra_l[pl.ds(64 * b1, 64), :].astype(f32)
        )
        za_r.wait_recv()
        zsb_r[...] = (
            p_ref[pl.ds(send_b, 64), pl.ds(HC, HC)]
            + zra_r[pl.ds(64 * (1 - b1), 64), :].astype(f32)
        ).astype(bf16)
        zb_r = rdma(zsb_r, zrb_r, z_snd.at[3], z_rcv.at[3], dz2)
        zb_r.start()
        p_ref[pl.ds(keep_b, 64), pl.ds(HC, HC)] += (
            zra_r[pl.ds(64 * b1, 64), :].astype(f32)
        )

        def ag(rows, off, c0, ssem, rsem, dev):
            d = rdma(out_ref.at[pl.ds(rows + off, 64), pl.ds(c0, HC)],
                     out_ref.at[pl.ds(rows + off, 64), pl.ds(c0, HC)],
                     ssem, rsem, dev)
            d.start()
            return d

        o1 = 128 * b0 + 64 * b1
        o2 = 128 * b0 + 64 * (1 - b1)
        o3 = 128 * (1 - b0) + 64 * b1
        o4 = 128 * (1 - b0) + 64 * (1 - b1)
        PIECES = (o1, o2, o3, o4)

        def zswap(rows, c0, k, dev):
            d = rdma(out_ref.at[pl.ds(rows, 64), pl.ds(c0, HC)],
                     out_ref.at[pl.ds(rows, 64), pl.ds(c0, HC)],
                     z_snd.at[k], z_rcv.at[k], dev)
            d.start()
            return d

        acw = [None] * 4
        accw = [None] * 4

        zb_l.wait_recv()
        out_ref[pl.ds(keep_b, 64), pl.ds(0, HC)] = (
            p_ref[pl.ds(keep_b, 64), pl.ds(0, HC)] + zrb_l[...].astype(f32)
        ).astype(bf16)
        acw[0] = ag(base, o1, 0, asnd_cw.at[0], arcv_cw.at[0, 0], right)
        zc_l = zswap(keep_b, 0, 4, dz2)
        zd1_l = zswap(keep_b, 0, 6, dz1)

        zb_r.wait_recv()
        out_ref[pl.ds(keep_b, 64), pl.ds(HC, HC)] = (
            p_ref[pl.ds(keep_b, 64), pl.ds(HC, HC)] + zrb_r[...].astype(f32)
        ).astype(bf16)
        accw[0] = ag(base, o1, HC, asnd_ccw.at[0], arcv_ccw.at[0, 0], left)
        zc_r = zswap(keep_b, HC, 5, dz2)
        zd1_r = zswap(keep_b, HC, 7, dz1)

        zc_l.wait_recv()
        acw[1] = ag(base, o2, 0, asnd_cw.at[1], arcv_cw.at[1, 0], right)
        zd2_l = zswap(base + o2, 0, 8, dz1)
        zc_r.wait_recv()
        accw[1] = ag(base, o2, HC, asnd_ccw.at[1], arcv_ccw.at[1, 0], left)
        zd2_r = zswap(base + o2, HC, 9, dz1)

        zd1_l.wait_recv()
        acw[2] = ag(base, o3, 0, asnd_cw.at[2], arcv_cw.at[2, 0], right)
        zd1_r.wait_recv()
        accw[2] = ag(base, o3, HC, asnd_ccw.at[2], arcv_ccw.at[2, 0], left)

        zd2_l.wait_recv()
        acw[3] = ag(base, o4, 0, asnd_cw.at[3], arcv_cw.at[3, 0], right)
        zd2_r.wait_recv()
        accw[3] = ag(base, o4, HC, asnd_ccw.at[3], arcv_ccw.at[3, 0], left)

        for h in range(NP - 1):
            n_cw = lax.rem(qi - h + NP, NP) * MC
            n_ccw = lax.rem(qi + h + 2, NP) * MC
            for k, off in enumerate(PIECES):
                acw[k].wait_recv()
                accw[k].wait_recv()
                if h < NP - 2:
                    acw[k].wait_send()
                    acw[k] = ag(n_cw, off, 0, asnd_cw.at[k],
                                arcv_cw.at[k, h + 1], right)
                    accw[k].wait_send()
                    accw[k] = ag(n_ccw, off, HC, asnd_ccw.at[k],
                                 arcv_ccw.at[k, h + 1], left)

        for d in (*cwd, *ccwd, za_l, za_r, zb_l, zb_r,
                  zc_l, zc_r, zd1_l, zd1_r, zd2_l, zd2_r, *acw, *accw):
            d.wait_send()

        @functools.partial(pl.run_scoped, exit_sem=pltpu.SemaphoreType.REGULAR)
        def _(exit_sem):
            for nbr in (left, right, dz1, dz2):
                xsig(exit_sem, nbr)
            pl.semaphore_wait(exit_sem, 4)

    dma = pltpu.SemaphoreType.DMA
    return pl.pallas_call(
        body,
        out_shape=jax.ShapeDtypeStruct((M, N), bf16),
        in_specs=[
            pl.BlockSpec(memory_space=pltpu.VMEM),
            pl.BlockSpec(memory_space=pltpu.VMEM),
        ],
        out_specs=pl.BlockSpec(memory_space=pltpu.VMEM),
        scratch_shapes=[
            pltpu.VMEM((M, N), f32),
            pltpu.VMEM((A.shape[1], N), bf16),
            pltpu.VMEM((MC, HC), bf16),
            pltpu.VMEM((MC, HC), bf16),
            pltpu.VMEM((NP - 1, MC, HC), bf16),
            pltpu.VMEM((NP - 1, MC, HC), bf16),
            pltpu.VMEM((HF, HC), bf16),
            pltpu.VMEM((HF, HC), bf16),
            pltpu.VMEM((HF, HC), bf16),
            pltpu.VMEM((HF, HC), bf16),
            pltpu.VMEM((64, HC), bf16),
            pltpu.VMEM((64, HC), bf16),
            pltpu.VMEM((64, HC), bf16),
            pltpu.VMEM((64, HC), bf16),
            dma((4,)), dma((4,)),
            dma((4, NP - 1)), dma((4, NP - 1)),
            dma((10,)), dma((10,)),
            dma((4,)), dma((4,)),
            dma((4, NP - 1)), dma((4, NP - 1)),
        ],
        compiler_params=pltpu.CompilerParams(collective_id=0),
    )(A, B)


# device time: 39842 ns/iter; 1.2295x vs baseline; 1.0325x over previous
import functools

import jax
import jax.numpy as jnp
from jax import lax
from jax.experimental import pallas as pl
from jax.experimental.pallas import tpu as pltpu

N_DEV = 16
NP = 4
MC = 256
HF = 128
HC = 512


def kernel(A, B):
    M, _ = A.shape
    _, N = B.shape

    f32 = jnp.float32
    bf16 = jnp.bfloat16

    def body(
        a_ref, b_ref, out_ref,
        p_ref, bb, sb_cw, sb_ccw, rb_cw, rb_ccw,
        zsa_l, zsa_r, zra_l, zra_r, zsb_l, zsb_r, zrb_l, zrb_r,
        snd_cw, snd_ccw, rcv_cw, rcv_ccw,
        z_snd, z_rcv,
        asnd_cw, asnd_ccw, arcv_cw, arcv_ccw,
    ):
        my = lax.axis_index("i")
        qi = lax.rem(my, NP)
        zi = lax.div(my, NP)
        b0 = lax.rem(zi, 2)
        b1 = lax.div(zi, 2)

        right = zi * NP + lax.rem(qi + 1, NP)
        left = zi * NP + lax.rem(qi + 3, NP)
        dz1 = (zi + 1 - 2 * b0) * NP + qi
        dz2 = (zi + 2 - 4 * b1) * NP + qi

        def xsig(sem, dev):
            pl.semaphore_signal(
                sem, inc=1, device_id=(dev,),
                device_id_type=pl.DeviceIdType.MESH,
            )

        barrier_sem = pltpu.get_barrier_semaphore()
        for nbr in (left, right, dz1, dz2):
            xsig(barrier_sem, nbr)
        pl.semaphore_wait(barrier_sem, 4)

        def rdma(src, dst, ssem, rsem, dev):
            return pltpu.make_async_remote_copy(
                src_ref=src, dst_ref=dst, send_sem=ssem, recv_sem=rsem,
                device_id=(dev,), device_id_type=pl.DeviceIdType.MESH,
            )

        def mm(ck, c0):
            rows = lax.rem(ck + NP, NP) * MC
            d = jnp.dot(
                a_ref[pl.ds(rows, MC), :].astype(bf16),
                bb[:, pl.ds(c0, HC)],
                preferred_element_type=f32,
            )
            p_ref[pl.ds(rows, MC), pl.ds(c0, HC)] = d
            return d

        POFF = (
            128 * (1 - b0) + 64 * (1 - b1),
            128 * (1 - b0) + 64 * b1,
            128 * b0,
            128 * b0 + 64,
        )

        def rs_start(sb, rb, rsems, dev):
            ds = []
            for k, off in enumerate(POFF):
                d = rdma(sb.at[pl.ds(off, 64), :], rb.at[0, pl.ds(off, 64), :],
                         (snd_cw if dev is right else snd_ccw).at[k],
                         rsems.at[k, 0], dev)
                d.start()
                ds.append(d)
            return ds

        bb[:, pl.ds(0, HC)] = b_ref[:, pl.ds(0, HC)].astype(bf16)
        sb_cw[...] = mm(qi, 0).astype(bf16)
        cwd = rs_start(sb_cw, rb_cw, rcv_cw, right)

        bb[:, pl.ds(HC, HC)] = b_ref[:, pl.ds(HC, HC)].astype(bf16)
        sb_ccw[...] = mm(qi + 2, HC).astype(bf16)
        ccwd = rs_start(sb_ccw, rb_ccw, rcv_ccw, left)

        mm(qi - 1, 0)
        mm(qi + 3, HC)
        mm(qi - 2, 0)
        mm(qi, HC)
        mm(qi + 1, 0)
        mm(qi + 1, HC)

        base = lax.rem(qi + 1, NP) * MC

        for h in range(NP - 2):
            cw_r = lax.rem(qi - h + 3, NP) * MC
            ccw_r = lax.rem(qi + h + 3, NP) * MC

            def fwd(desc, sb, rb, rows, c0, k, ssems, rsems, dev):
                off = POFF[k]
                desc.wait_recv()
                desc.wait_send()
                sb[pl.ds(off, 64), :] = (
                    p_ref[pl.ds(rows + off, 64), pl.ds(c0, HC)]
                    + rb[h, pl.ds(off, 64), :].astype(f32)
                ).astype(bf16)
                nxt = rdma(sb.at[pl.ds(off, 64), :],
                           rb.at[h + 1, pl.ds(off, 64), :],
                           ssems.at[k], rsems.at[k, h + 1], dev)
                nxt.start()
                return nxt

            for k in range(4):
                cwd[k] = fwd(cwd[k], sb_cw, rb_cw, cw_r, 0, k,
                             snd_cw, rcv_cw, right)
                ccwd[k] = fwd(ccwd[k], sb_ccw, rb_ccw, ccw_r, HC, k,
                              snd_ccw, rcv_ccw, left)

        keep_a = base + 128 * b0
        send_a = base + 128 * (1 - b0)
        hi = 64 * (1 - b1)
        lo = 64 * b1
        H = NP - 2

        def za_piece(zsa, zra, c0, rbuf, off, k):
            zsa[pl.ds(off, 64), :] = (
                p_ref[pl.ds(send_a + off, 64), pl.ds(c0, HC)]
                + rbuf[H, pl.ds(128 * (1 - b0) + off, 64), :].astype(f32)
            ).astype(bf16)
            d = rdma(zsa.at[pl.ds(off, 64), :], zra.at[pl.ds(off, 64), :],
                     z_snd.at[k], z_rcv.at[k], dz1)
            d.start()
            return d

        cwd[0].wait_recv()
        za_l_hi = za_piece(zsa_l, zra_l, 0, rb_cw, hi, 0)
        cwd[1].wait_recv()
        za_l_lo = za_piece(zsa_l, zra_l, 0, rb_cw, lo, 10)
        ccwd[0].wait_recv()
        za_r_hi = za_piece(zsa_r, zra_r, HC, rb_ccw, hi, 1)
        ccwd[1].wait_recv()
        za_r_lo = za_piece(zsa_r, zra_r, HC, rb_ccw, lo, 11)

        cwd[2].wait_recv()
        cwd[3].wait_recv()
        p_ref[pl.ds(keep_a, HF), pl.ds(0, HC)] += (
            rb_cw[H, pl.ds(128 * b0, HF), :].astype(f32)
        )
        ccwd[2].wait_recv()
        ccwd[3].wait_recv()
        p_ref[pl.ds(keep_a, HF), pl.ds(HC, HC)] += (
            rb_ccw[H, pl.ds(128 * b0, HF), :].astype(f32)
        )

        keep_b = keep_a + 64 * b1
        send_b = keep_a + 64 * (1 - b1)
        za_l_hi.wait_recv()
        zsb_l[...] = (
            p_ref[pl.ds(send_b, 64), pl.ds(0, HC)]
            + zra_l[pl.ds(64 * (1 - b1), 64), :].astype(f32)
        ).astype(bf16)
        zb_l = rdma(zsb_l, zrb_l, z_snd.at[2], z_rcv.at[2], dz2)
        zb_l.start()
        za_r_hi.wait_recv()
        zsb_r[...] = (
            p_ref[pl.ds(send_b, 64), pl.ds(HC, HC)]
            + zra_r[pl.ds(64 * (1 - b1), 64), :].astype(f32)
        ).astype(bf16)
        zb_r = rdma(zsb_r, zrb_r, z_snd.at[3], z_rcv.at[3], dz2)
        zb_r.start()
        za_l_lo.wait_recv()
        p_ref[pl.ds(keep_b, 64), pl.ds(0, HC)] += (
            zra_l[pl.ds(64 * b1, 64), :].astype(f32)
        )
        za_r_lo.wait_recv()
        p_ref[pl.ds(keep_b, 64), pl.ds(HC, HC)] += (
            zra_r[pl.ds(64 * b1, 64), :].astype(f32)
        )

        def ag(rows, off, c0, ssem, rsem, dev):
            d = rdma(out_ref.at[pl.ds(rows + off, 64), pl.ds(c0, HC)],
                     out_ref.at[pl.ds(rows + off, 64), pl.ds(c0, HC)],
                     ssem, rsem, dev)
            d.start()
            return d

        o1 = 128 * b0 + 64 * b1
        o2 = 128 * b0 + 64 * (1 - b1)
        o3 = 128 * (1 - b0) + 64 * b1
        o4 = 128 * (1 - b0) + 64 * (1 - b1)
        PIECES = (o1, o2, o3, o4)

        def zswap(rows, c0, k, dev):
            d = rdma(out_ref.at[pl.ds(rows, 64), pl.ds(c0, HC)],
                     out_ref.at[pl.ds(rows, 64), pl.ds(c0, HC)],
                     z_snd.at[k], z_rcv.at[k], dev)
            d.start()
            return d

        acw = [None] * 4
        accw = [None] * 4

        zb_l.wait_recv()
        out_ref[pl.ds(keep_b, 64), pl.ds(0, HC)] = (
            p_ref[pl.ds(keep_b, 64), pl.ds(0, HC)] + zrb_l[...].astype(f32)
        ).astype(bf16)
        acw[0] = ag(base, o1, 0, asnd_cw.at[0], arcv_cw.at[0, 0], right)
        zc_l = zswap(keep_b, 0, 4, dz2)
        zd1_l = zswap(keep_b, 0, 6, dz1)

        zb_r.wait_recv()
        out_ref[pl.ds(keep_b, 64), pl.ds(HC, HC)] = (
            p_ref[pl.ds(keep_b, 64), pl.ds(HC, HC)] + zrb_r[...].astype(f32)
        ).astype(bf16)
        accw[0] = ag(base, o1, HC, asnd_ccw.at[0], arcv_ccw.at[0, 0], left)
        zc_r = zswap(keep_b, HC, 5, dz2)
        zd1_r = zswap(keep_b, HC, 7, dz1)

        zc_l.wait_recv()
        acw[1] = ag(base, o2, 0, asnd_cw.at[1], arcv_cw.at[1, 0], right)
        zd2_l = zswap(base + o2, 0, 8, dz1)
        zc_r.wait_recv()
        accw[1] = ag(base, o2, HC, asnd_ccw.at[1], arcv_ccw.at[1, 0], left)
        zd2_r = zswap(base + o2, HC, 9, dz1)

        zd1_l.wait_recv()
        acw[2] = ag(base, o3, 0, asnd_cw.at[2], arcv_cw.at[2, 0], right)
        zd1_r.wait_recv()
        accw[2] = ag(base, o3, HC, asnd_ccw.at[2], arcv_ccw.at[2, 0], left)

        zd2_l.wait_recv()
        acw[3] = ag(base, o4, 0, asnd_cw.at[3], arcv_cw.at[3, 0], right)
        zd2_r.wait_recv()
        accw[3] = ag(base, o4, HC, asnd_ccw.at[3], arcv_ccw.at[3, 0], left)

        for h in range(NP - 1):
            n_cw = lax.rem(qi - h + NP, NP) * MC
            n_ccw = lax.rem(qi + h + 2, NP) * MC
            for k, off in enumerate(PIECES):
                acw[k].wait_recv()
                accw[k].wait_recv()
                if h < NP - 2:
                    acw[k].wait_send()
                    acw[k] = ag(n_cw, off, 0, asnd_cw.at[k],
                                arcv_cw.at[k, h + 1], right)
                    accw[k].wait_send()
                    accw[k] = ag(n_ccw, off, HC, asnd_ccw.at[k],
                                 arcv_ccw.at[k, h + 1], left)

        for d in (*cwd, *ccwd, za_l_hi, za_l_lo, za_r_hi, za_r_lo,
                  zb_l, zb_r, zc_l, zc_r, zd1_l, zd1_r, zd2_l, zd2_r,
                  *acw, *accw):
            d.wait_send()

        @functools.partial(pl.run_scoped, exit_sem=pltpu.SemaphoreType.REGULAR)
        def _(exit_sem):
            for nbr in (left, right, dz1, dz2):
                xsig(exit_sem, nbr)
            pl.semaphore_wait(exit_sem, 4)

    dma = pltpu.SemaphoreType.DMA
    return pl.pallas_call(
        body,
        out_shape=jax.ShapeDtypeStruct((M, N), bf16),
        in_specs=[
            pl.BlockSpec(memory_space=pltpu.VMEM),
            pl.BlockSpec(memory_space=pltpu.VMEM),
        ],
        out_specs=pl.BlockSpec(memory_space=pltpu.VMEM),
        scratch_shapes=[
            pltpu.VMEM((M, N), f32),
            pltpu.VMEM((A.shape[1], N), bf16),
            pltpu.VMEM((MC, HC), bf16),
            pltpu.VMEM((MC, HC), bf16),
            pltpu.VMEM((NP - 1, MC, HC), bf16),
            pltpu.VMEM((NP - 1, MC, HC), bf16),
            pltpu.VMEM((HF, HC), bf16),
            pltpu.VMEM((HF, HC), bf16),
            pltpu.VMEM((HF, HC), bf16),
            pltpu.VMEM((HF, HC), bf16),
            pltpu.VMEM((64, HC), bf16),
            pltpu.VMEM((64, HC), bf16),
            pltpu.VMEM((64, HC), bf16),
            pltpu.VMEM((64, HC), bf16),
            dma((4,)), dma((4,)),
            dma((4, NP - 1)), dma((4, NP - 1)),
            dma((12,)), dma((12,)),
            dma((4,)), dma((4,)),
            dma((4, NP - 1)), dma((4, NP - 1)),
        ],
        compiler_params=pltpu.CompilerParams(collective_id=0),
    )(A, B)
